# Initial kernel scaffold; baseline (speedup 1.0000x reference)
import jax
import jax.numpy as jnp
from jax.experimental import pallas as pl


def kernel(idx, code):
    raise NotImplementedError("write your pallas kernel here")


# speedup vs baseline: 2.6452x; 2.6452x over previous
import functools

import jax
import jax.numpy as jnp
from jax import lax
from jax.experimental import pallas as pl
from jax.experimental.pallas import tpu as pltpu
from jax.experimental.pallas import tpu_sc as plsc

NC = 2
NS = 16
NW = NC * NS

B = 4096
D = 8192
BPW = B // NW
CK = 8


def _gather_body(idx_hbm, table_hbm, out_hbm, idx_v, buf, sem):
    wid = lax.axis_index("s") * NC + lax.axis_index("c")
    base = wid * BPW
    pltpu.sync_copy(idx_hbm.at[pl.ds(base, BPW)], idx_v)

    def chunk(g, carry):
        off = g * CK
        pltpu.async_copy(
            table_hbm.at[idx_v.at[pl.ds(off, CK)]], buf, sem
        ).wait()
        pltpu.sync_copy(buf, out_hbm.at[pl.ds(base + off, CK)])
        return carry

    lax.fori_loop(0, BPW // CK, chunk, 0, unroll=False)


@functools.partial(jax.jit, static_argnames=())
def _gather(idx, table):
    mesh = plsc.VectorSubcoreMesh(
        core_axis_name="c", subcore_axis_name="s", num_cores=NC, num_subcores=NS
    )
    return pl.kernel(
        _gather_body,
        out_type=jax.ShapeDtypeStruct((B, D), jnp.float32),
        mesh=mesh,
        scratch_types=[
            pltpu.VMEM((BPW,), jnp.int32),
            pltpu.VMEM((CK, D), jnp.float32),
            pltpu.SemaphoreType.DMA,
        ],
    )(idx, table)


def kernel(idx, code):
    n, c, h, w = code.shape
    table = code.reshape(n, c * h * w)
    out = _gather(idx.astype(jnp.int32), table)
    return out.reshape(-1, c, h, w)


# speedup vs baseline: 2.6812x; 1.0136x over previous
import functools

import jax
import jax.numpy as jnp
from jax import lax
from jax.experimental import pallas as pl
from jax.experimental.pallas import tpu as pltpu
from jax.experimental.pallas import tpu_sc as plsc

NC = 2
NS = 16
NW = NC * NS

B = 4096
D = 8192
BPW = B // NW
CK = 4
G = BPW // CK
G2 = G // 2


def _gather_body(idx_hbm, table_hbm, out_hbm,
                 idx_v, buf0, buf1, sg0, sg1, ss0, ss1):
    wid = lax.axis_index("s") * NC + lax.axis_index("c")
    base = wid * BPW
    pltpu.sync_copy(idx_hbm.at[wid], idx_v)

    def gather(g, buf, sem):
        return pltpu.make_async_copy(table_hbm.at[idx_v.at[g]], buf, sem)

    def store(g, buf, sem):
        return pltpu.make_async_copy(buf, out_hbm.at[pl.ds(base + g * CK, CK)], sem)

    gather(0, buf0, sg0).start()

    def body(p, carry):
        g0 = 2 * p
        g1 = g0 + 1

        @pl.when(p > 0)
        def _():
            store(g1 - 2, buf1, ss1).wait()

        gather(g1, buf1, sg1).start()
        gather(g0, buf0, sg0).wait()
        store(g0, buf0, ss0).start()

        @pl.when(p < G2 - 1)
        def _():
            store(g0, buf0, ss0).wait()
            gather(g0 + 2, buf0, sg0).start()

        gather(g1, buf1, sg1).wait()
        store(g1, buf1, ss1).start()
        return carry

    lax.fori_loop(0, G2, body, 0, unroll=False)
    store(G - 2, buf0, ss0).wait()
    store(G - 1, buf1, ss1).wait()


@jax.jit
def _gather(idx3, table):
    mesh = plsc.VectorSubcoreMesh(
        core_axis_name="c", subcore_axis_name="s", num_cores=NC, num_subcores=NS
    )
    return pl.kernel(
        _gather_body,
        out_type=jax.ShapeDtypeStruct((B, D), jnp.float32),
        mesh=mesh,
        scratch_types=[
            pltpu.VMEM((G, CK), jnp.int32),
            pltpu.VMEM((CK, D), jnp.float32),
            pltpu.VMEM((CK, D), jnp.float32),
            pltpu.SemaphoreType.DMA,
            pltpu.SemaphoreType.DMA,
            pltpu.SemaphoreType.DMA,
            pltpu.SemaphoreType.DMA,
        ],
    )(idx3, table)


def kernel(idx, code):
    n, c, h, w = code.shape
    table = code.reshape(n, c * h * w)
    idx3 = idx.astype(jnp.int32).reshape(NW, G, CK)
    out = _gather(idx3, table)
    return out.reshape(-1, c, h, w)
